# baseline (speedup 1.0000x reference)
import jax
import jax.numpy as jnp
from jax import lax
from jax.experimental import pallas as pl
from jax.experimental.pallas import tpu as pltpu
from jax.experimental.pallas import tpu_sc as plsc

B = 16
T = 2048
D = 64
NC = 2
NS = 16
NW = NC * NS
NB = 8
TW = 128
LANES = 16
VPD = D // LANES


def _emb_body(x_hbm, wte_hbm, wpe_hbm, out_hbm, idx_v, rows_v, pos_v, sem):
    wid = lax.axis_index("s") * NC + lax.axis_index("c")
    b0 = (wid % 2) * NB
    t0 = (wid // 2) * TW

    pltpu.sync_copy(x_hbm.at[pl.ds(b0, NB), pl.ds(t0, TW)], idx_v)
    pltpu.sync_copy(wpe_hbm.at[pl.ds(t0, TW)], pos_v)

    gathers = [
        pltpu.async_copy(wte_hbm.at[idx_v.at[b]], rows_v.at[b], sem)
        for b in range(NB)
    ]
    for g in gathers:
        g.wait()

    def add_t(t, carry):
        for v in range(VPD):
            p = pos_v[t, pl.ds(v * LANES, LANES)]
            for b in range(NB):
                plsc.addupdate(rows_v.at[b, t, pl.ds(v * LANES, LANES)], p)
        return carry

    lax.fori_loop(0, TW, add_t, 0)

    outs = [
        pltpu.async_copy(
            rows_v.at[b], out_hbm.at[b0 + b, pl.ds(t0, TW), :], sem
        )
        for b in range(NB)
    ]
    for o in outs:
        o.wait()


@jax.jit
def kernel(x, wte, wpe):
    run = pl.kernel(
        _emb_body,
        out_type=jax.ShapeDtypeStruct((B, T, D), jnp.float32),
        mesh=plsc.VectorSubcoreMesh(core_axis_name="c", subcore_axis_name="s"),
        scratch_types=[
            pltpu.VMEM((NB, TW), jnp.int32),
            pltpu.VMEM((NB, TW, D), jnp.float32),
            pltpu.VMEM((TW, D), jnp.float32),
            pltpu.SemaphoreType.DMA,
        ],
        compiler_params=pltpu.CompilerParams(use_tc_tiling_on_sc=False),
    )
    return run(x, wte, wpe)
